# baseline (device time: 376638 ns/iter reference)
import jax
import jax.numpy as jnp
from jax import lax
from jax.experimental import pallas as pl
from jax.experimental.pallas import tpu as pltpu

N_RING = 8


def _peer():
    return (lax.axis_index("x"), 1 - lax.axis_index("y"), lax.axis_index("z"))


def _ring_pos():
    x = lax.axis_index("x")
    z = lax.axis_index("z")
    return jnp.where(x == 0, z, N_RING - 1 - z)


def _ring_coords(t):
    x = jnp.where(t < N_RING // 2, 0, 1)
    z = jnp.where(t < N_RING // 2, t, N_RING - 1 - t)
    return x, z


def _exchange_kernel(xsl, asl):
    S, D = xsl.shape

    def body(x_ref, a_ref, xloc_ref, aloc_ref, sx_send, sx_recv, sa_send, sa_recv):
        peer = _peer()
        xloc_ref[0] = x_ref[...]
        aloc_ref[0] = a_ref[...]
        rx = pltpu.make_async_remote_copy(
            src_ref=x_ref,
            dst_ref=xloc_ref.at[1],
            send_sem=sx_send,
            recv_sem=sx_recv,
            device_id=peer,
            device_id_type=pl.DeviceIdType.MESH,
        )
        ra = pltpu.make_async_remote_copy(
            src_ref=a_ref,
            dst_ref=aloc_ref.at[1],
            send_sem=sa_send,
            recv_sem=sa_recv,
            device_id=peer,
            device_id_type=pl.DeviceIdType.MESH,
        )
        rx.start()
        ra.start()
        rx.wait()
        ra.wait()

    return pl.pallas_call(
        body,
        out_shape=(
            jax.ShapeDtypeStruct((2, S, D), jnp.bfloat16),
            jax.ShapeDtypeStruct((2, S, 1), jnp.int32),
        ),
        in_specs=[
            pl.BlockSpec(memory_space=pltpu.MemorySpace.VMEM),
            pl.BlockSpec(memory_space=pltpu.MemorySpace.VMEM),
        ],
        out_specs=(
            pl.BlockSpec(memory_space=pltpu.MemorySpace.VMEM),
            pl.BlockSpec(memory_space=pltpu.MemorySpace.VMEM),
        ),
        scratch_shapes=[
            pltpu.SemaphoreType.DMA,
            pltpu.SemaphoreType.DMA,
            pltpu.SemaphoreType.DMA,
            pltpu.SemaphoreType.DMA,
        ],
    )(xsl, asl)


def _moe_combine_allgather_kernel(xloc, aloc, W1f, W2f, FT=512):
    _, S, D = xloc.shape
    E_loc, _, F = W1f.shape
    n_f = F // FT
    n_r = N_RING // 2
    n_l = N_RING - 1 - n_r
    T = N_RING * S

    def body(
        x_ref,
        a_ref,
        w1_ref,
        w2_ref,
        out_ref,
        acc_ref,
        xm_ref,
        csend_ref,
        crecv_ref,
        commr_ref,
        comml_ref,
        c_send,
        c_recv,
        sr_send,
        sr_recv,
        sl_send,
        sl_recv,
    ):
        m = pl.program_id(0)
        e = pl.program_id(1)
        f = pl.program_id(2)
        my_y = lax.axis_index("y")
        peer = _peer()
        ge = my_y * E_loc + e

        def c_rdma():
            return pltpu.make_async_remote_copy(
                src_ref=csend_ref,
                dst_ref=crecv_ref,
                send_sem=c_send,
                recv_sem=c_recv,
                device_id=peer,
                device_id_type=pl.DeviceIdType.MESH,
            )

        @pl.when(f == 0)
        def _():
            mask = a_ref[0] == ge
            xm_ref[...] = jnp.where(mask, x_ref[0], jnp.bfloat16(0))

        w1 = w1_ref[0].astype(jnp.bfloat16)
        w2 = w2_ref[0].astype(jnp.bfloat16)
        h = jnp.dot(xm_ref[...], w1, preferred_element_type=jnp.float32)
        h = jnp.maximum(h, 0.0).astype(jnp.bfloat16)
        part = jnp.dot(h, w2, preferred_element_type=jnp.float32)

        first = jnp.logical_and(e == 0, f == 0)
        last_ef = jnp.logical_and(e == E_loc - 1, f == n_f - 1)

        @pl.when(first)
        def _():
            acc_ref[...] = part

        @pl.when(jnp.logical_not(first))
        def _():
            acc_ref[...] += part

        @pl.when(jnp.logical_and(m == 0, last_ef))
        def _():
            csend_ref[...] = acc_ref[...].astype(jnp.bfloat16)
            c_rdma().start()

        @pl.when(jnp.logical_and(m == 1, last_ef))
        def _():
            rp = _ring_pos()
            rx, rz = _ring_coords((rp + 1) % N_RING)
            lx, lz = _ring_coords((rp - 1) % N_RING)
            right = (rx, my_y, rz)
            left = (lx, my_y, lz)

            rc = c_rdma()
            rc.wait_send()
            rc.wait_recv()
            own = acc_ref[...].astype(jnp.bfloat16) + crecv_ref[...]

            out_ref[pl.ds(rp * S, S), :] = own
            commr_ref[0] = own
            comml_ref[0] = own

            for hop in range(n_r):
                sslot = hop % 2
                rslot = (hop + 1) % 2
                rdma_r = pltpu.make_async_remote_copy(
                    src_ref=commr_ref.at[sslot],
                    dst_ref=commr_ref.at[rslot],
                    send_sem=sr_send.at[sslot],
                    recv_sem=sr_recv.at[rslot],
                    device_id=right,
                    device_id_type=pl.DeviceIdType.MESH,
                )
                rdma_r.start()
                if hop < n_l:
                    rdma_l = pltpu.make_async_remote_copy(
                        src_ref=comml_ref.at[sslot],
                        dst_ref=comml_ref.at[rslot],
                        send_sem=sl_send.at[sslot],
                        recv_sem=sl_recv.at[rslot],
                        device_id=left,
                        device_id_type=pl.DeviceIdType.MESH,
                    )
                    rdma_l.start()
                rdma_r.wait()
                origin_r = (rp - hop - 1) % N_RING
                out_ref[pl.ds(origin_r * S, S), :] = commr_ref[rslot]
                if hop < n_l:
                    rdma_l.wait()
                    origin_l = (rp + hop + 1) % N_RING
                    out_ref[pl.ds(origin_l * S, S), :] = comml_ref[rslot]

    return pl.pallas_call(
        body,
        grid=(2, E_loc, n_f),
        out_shape=jax.ShapeDtypeStruct((T, D), jnp.bfloat16),
        in_specs=[
            pl.BlockSpec((1, S, D), lambda m, e, f: (1 - m, 0, 0)),
            pl.BlockSpec((1, S, 1), lambda m, e, f: (1 - m, 0, 0)),
            pl.BlockSpec((1, D, FT), lambda m, e, f: (e, 0, f)),
            pl.BlockSpec((1, FT, D), lambda m, e, f: (e, f, 0)),
        ],
        out_specs=pl.BlockSpec((T, D), lambda m, e, f: (0, 0)),
        scratch_shapes=[
            pltpu.VMEM((S, D), jnp.float32),
            pltpu.VMEM((S, D), jnp.bfloat16),
            pltpu.VMEM((S, D), jnp.bfloat16),
            pltpu.VMEM((S, D), jnp.bfloat16),
            pltpu.VMEM((2, S, D), jnp.bfloat16),
            pltpu.VMEM((2, S, D), jnp.bfloat16),
            pltpu.SemaphoreType.DMA,
            pltpu.SemaphoreType.DMA,
            pltpu.SemaphoreType.DMA((2,)),
            pltpu.SemaphoreType.DMA((2,)),
            pltpu.SemaphoreType.DMA((2,)),
            pltpu.SemaphoreType.DMA((2,)),
        ],
        compiler_params=pltpu.CompilerParams(
            dimension_semantics=("arbitrary", "arbitrary", "arbitrary"),
            vmem_limit_bytes=62 * 2**20,
        ),
    )(xloc, aloc, W1f, W2f)


def kernel(x, assign, W1, W2):
    T, D = x.shape
    S = T // N_RING
    xb = x.astype(jnp.bfloat16)
    a2 = assign.reshape(T, 1)

    rp = _ring_pos()
    xsl = lax.dynamic_slice(xb, (rp * S, 0), (S, D))
    asl = lax.dynamic_slice(a2, (rp * S, 0), (S, 1))

    xloc, aloc = _exchange_kernel(xsl, asl)
    return _moe_combine_allgather_kernel(xloc, aloc, W1, W2)


# device time: 368742 ns/iter; 1.0214x vs baseline; 1.0214x over previous
import jax
import jax.numpy as jnp
from jax import lax
from jax.experimental import pallas as pl
from jax.experimental.pallas import tpu as pltpu

N_RING = 8


def _peer():
    return (lax.axis_index("x"), 1 - lax.axis_index("y"), lax.axis_index("z"))


def _ring_pos():
    x = lax.axis_index("x")
    z = lax.axis_index("z")
    return jnp.where(x == 0, z, N_RING - 1 - z)


def _ring_coords(t):
    x = jnp.where(t < N_RING // 2, 0, 1)
    z = jnp.where(t < N_RING // 2, t, N_RING - 1 - t)
    return x, z


def _exchange_kernel(xsl, asl):
    S, D = xsl.shape

    def body(x_ref, a_ref, xloc_ref, aloc_ref, sx_send, sx_recv, sa_send, sa_recv):
        peer = _peer()
        xloc_ref[0] = x_ref[...]
        aloc_ref[0] = a_ref[...]
        rx = pltpu.make_async_remote_copy(
            src_ref=x_ref,
            dst_ref=xloc_ref.at[1],
            send_sem=sx_send,
            recv_sem=sx_recv,
            device_id=peer,
            device_id_type=pl.DeviceIdType.MESH,
        )
        ra = pltpu.make_async_remote_copy(
            src_ref=a_ref,
            dst_ref=aloc_ref.at[1],
            send_sem=sa_send,
            recv_sem=sa_recv,
            device_id=peer,
            device_id_type=pl.DeviceIdType.MESH,
        )
        rx.start()
        ra.start()
        rx.wait()
        ra.wait()

    return pl.pallas_call(
        body,
        out_shape=(
            jax.ShapeDtypeStruct((2, S, D), jnp.bfloat16),
            jax.ShapeDtypeStruct((2, S, 1), jnp.int32),
        ),
        in_specs=[
            pl.BlockSpec(memory_space=pltpu.MemorySpace.VMEM),
            pl.BlockSpec(memory_space=pltpu.MemorySpace.VMEM),
        ],
        out_specs=(
            pl.BlockSpec(memory_space=pltpu.MemorySpace.VMEM),
            pl.BlockSpec(memory_space=pltpu.MemorySpace.VMEM),
        ),
        scratch_shapes=[
            pltpu.SemaphoreType.DMA,
            pltpu.SemaphoreType.DMA,
            pltpu.SemaphoreType.DMA,
            pltpu.SemaphoreType.DMA,
        ],
    )(xsl, asl)


def _moe_combine_allgather_kernel(xloc, aloc, W1f, W2f, FT=512):
    n_m, S, D = xloc.shape
    E_loc, _, F = W1f.shape
    n_f = F // FT
    n_r = N_RING // 2
    n_l = N_RING - 1 - n_r
    T = N_RING * S
    R = n_m * S

    def body(
        x_ref,
        a_ref,
        w1_ref,
        w2_ref,
        out_ref,
        acc_ref,
        xm_ref,
        csend_ref,
        crecv_ref,
        commr_ref,
        comml_ref,
        c_send,
        c_recv,
        sr_send,
        sr_recv,
        sl_send,
        sl_recv,
        st_sems,
    ):
        e = pl.program_id(0)
        f = pl.program_id(1)
        my_y = lax.axis_index("y")
        peer = _peer()
        ge = my_y * E_loc + e

        @pl.when(f == 0)
        def _():
            mask = a_ref[...].reshape(R, 1) == ge
            xm_ref[...] = jnp.where(mask, x_ref[...].reshape(R, D), jnp.bfloat16(0))

        w1 = w1_ref[0].astype(jnp.bfloat16)
        w2 = w2_ref[0].astype(jnp.bfloat16)
        h = jnp.dot(xm_ref[...], w1, preferred_element_type=jnp.float32)
        h = jnp.maximum(h, 0.0).astype(jnp.bfloat16)
        part = jnp.dot(h, w2, preferred_element_type=jnp.float32)

        first = jnp.logical_and(e == 0, f == 0)
        last_ef = jnp.logical_and(e == E_loc - 1, f == n_f - 1)

        @pl.when(first)
        def _():
            acc_ref[...] = part

        @pl.when(jnp.logical_not(first))
        def _():
            acc_ref[...] += part

        @pl.when(last_ef)
        def _():
            rp = _ring_pos()
            rx, rz = _ring_coords((rp + 1) % N_RING)
            lx, lz = _ring_coords((rp - 1) % N_RING)
            right = (rx, my_y, rz)
            left = (lx, my_y, lz)

            csend_ref[...] = acc_ref[pl.ds(S, S), :].astype(jnp.bfloat16)
            rc = pltpu.make_async_remote_copy(
                src_ref=csend_ref,
                dst_ref=crecv_ref,
                send_sem=c_send,
                recv_sem=c_recv,
                device_id=peer,
                device_id_type=pl.DeviceIdType.MESH,
            )
            rc.start()
            rc.wait()
            own = acc_ref[pl.ds(0, S), :].astype(jnp.bfloat16) + crecv_ref[...]

            commr_ref[0] = own
            comml_ref[0] = own
            own_store = pltpu.make_async_copy(
                commr_ref.at[0], out_ref.at[pl.ds(rp * S, S)], st_sems.at[rp]
            )
            own_store.start()

            stores = [own_store]
            for hop in range(n_r):
                sslot = hop % 2
                rslot = (hop + 1) % 2
                rdma_r = pltpu.make_async_remote_copy(
                    src_ref=commr_ref.at[sslot],
                    dst_ref=commr_ref.at[rslot],
                    send_sem=sr_send.at[sslot],
                    recv_sem=sr_recv.at[rslot],
                    device_id=right,
                    device_id_type=pl.DeviceIdType.MESH,
                )
                rdma_r.start()
                if hop < n_l:
                    rdma_l = pltpu.make_async_remote_copy(
                        src_ref=comml_ref.at[sslot],
                        dst_ref=comml_ref.at[rslot],
                        send_sem=sl_send.at[sslot],
                        recv_sem=sl_recv.at[rslot],
                        device_id=left,
                        device_id_type=pl.DeviceIdType.MESH,
                    )
                    rdma_l.start()
                rdma_r.wait()
                origin_r = (rp - hop - 1) % N_RING
                st_r = pltpu.make_async_copy(
                    commr_ref.at[rslot],
                    out_ref.at[pl.ds(origin_r * S, S)],
                    st_sems.at[origin_r],
                )
                st_r.start()
                stores.append(st_r)
                if hop < n_l:
                    rdma_l.wait()
                    origin_l = (rp + hop + 1) % N_RING
                    st_l = pltpu.make_async_copy(
                        comml_ref.at[rslot],
                        out_ref.at[pl.ds(origin_l * S, S)],
                        st_sems.at[origin_l],
                    )
                    st_l.start()
                    stores.append(st_l)
                if hop >= 1:
                    stores[2 * hop - 1].wait()
                    stores[2 * hop].wait()
            for st in (stores[0], stores[-1]):
                st.wait()

    return pl.pallas_call(
        body,
        grid=(E_loc, n_f),
        out_shape=jax.ShapeDtypeStruct((T, D), jnp.bfloat16),
        in_specs=[
            pl.BlockSpec((n_m, S, D), lambda e, f: (0, 0, 0)),
            pl.BlockSpec((n_m, S, 1), lambda e, f: (0, 0, 0)),
            pl.BlockSpec((1, D, FT), lambda e, f: (e, 0, f)),
            pl.BlockSpec((1, FT, D), lambda e, f: (e, f, 0)),
        ],
        out_specs=pl.BlockSpec(memory_space=pltpu.MemorySpace.HBM),
        scratch_shapes=[
            pltpu.VMEM((R, D), jnp.float32),
            pltpu.VMEM((R, D), jnp.bfloat16),
            pltpu.VMEM((S, D), jnp.bfloat16),
            pltpu.VMEM((S, D), jnp.bfloat16),
            pltpu.VMEM((2, S, D), jnp.bfloat16),
            pltpu.VMEM((2, S, D), jnp.bfloat16),
            pltpu.SemaphoreType.DMA,
            pltpu.SemaphoreType.DMA,
            pltpu.SemaphoreType.DMA((2,)),
            pltpu.SemaphoreType.DMA((2,)),
            pltpu.SemaphoreType.DMA((2,)),
            pltpu.SemaphoreType.DMA((2,)),
            pltpu.SemaphoreType.DMA((N_RING,)),
        ],
        compiler_params=pltpu.CompilerParams(
            dimension_semantics=("arbitrary", "arbitrary"),
            vmem_limit_bytes=62 * 2**20,
        ),
    )(xloc, aloc, W1f, W2f)


def kernel(x, assign, W1, W2):
    T, D = x.shape
    S = T // N_RING
    xb = x.astype(jnp.bfloat16)
    a2 = assign.reshape(T, 1)

    rp = _ring_pos()
    xsl = lax.dynamic_slice(xb, (rp * S, 0), (S, D))
    asl = lax.dynamic_slice(a2, (rp * S, 0), (S, 1))

    xloc, aloc = _exchange_kernel(xsl, asl)
    return _moe_combine_allgather_kernel(xloc, aloc, W1, W2)


# device time: 365031 ns/iter; 1.0318x vs baseline; 1.0102x over previous
import jax
import jax.numpy as jnp
from jax import lax
from jax.experimental import pallas as pl
from jax.experimental.pallas import tpu as pltpu

N_RING = 8


def _peer():
    return (lax.axis_index("x"), 1 - lax.axis_index("y"), lax.axis_index("z"))


def _ring_pos():
    x = lax.axis_index("x")
    z = lax.axis_index("z")
    return jnp.where(x == 0, z, 2 * N_RING // 2 - 1 - z)


def _ring_coords(t):
    x = jnp.where(t < N_RING // 2, 0, 1)
    z = jnp.where(t < N_RING // 2, t, N_RING - 1 - t)
    return x, z


def _exchange_kernel(xsl, asl):
    S, D = xsl.shape

    def body(x_ref, a_ref, xloc_ref, aloc_ref, sx_send, sx_recv, sa_send, sa_recv):
        peer = _peer()
        xloc_ref[0] = x_ref[...]
        aloc_ref[0] = a_ref[...]
        rx = pltpu.make_async_remote_copy(
            src_ref=x_ref,
            dst_ref=xloc_ref.at[1],
            send_sem=sx_send,
            recv_sem=sx_recv,
            device_id=peer,
            device_id_type=pl.DeviceIdType.MESH,
        )
        ra = pltpu.make_async_remote_copy(
            src_ref=a_ref,
            dst_ref=aloc_ref.at[1],
            send_sem=sa_send,
            recv_sem=sa_recv,
            device_id=peer,
            device_id_type=pl.DeviceIdType.MESH,
        )
        rx.start()
        ra.start()
        rx.wait()
        ra.wait()

    return pl.pallas_call(
        body,
        out_shape=(
            jax.ShapeDtypeStruct((2, S, D), jnp.bfloat16),
            jax.ShapeDtypeStruct((2, S, 1), jnp.int32),
        ),
        in_specs=[
            pl.BlockSpec(memory_space=pltpu.MemorySpace.VMEM),
            pl.BlockSpec(memory_space=pltpu.MemorySpace.VMEM),
        ],
        out_specs=(
            pl.BlockSpec(memory_space=pltpu.MemorySpace.VMEM),
            pl.BlockSpec(memory_space=pltpu.MemorySpace.VMEM),
        ),
        scratch_shapes=[
            pltpu.SemaphoreType.DMA,
            pltpu.SemaphoreType.DMA,
            pltpu.SemaphoreType.DMA,
            pltpu.SemaphoreType.DMA,
        ],
    )(xsl, asl)


def _moe_kernel(xall, aall, W1f, W2f, FT=1024):
    n_m, S, D = xall.shape
    E_loc, _, F = W1f.shape
    n_f = F // FT
    R = n_m * S

    def body(x_ref, a_ref, w1_ref, w2_ref, out_ref, acc_ref, xm_ref):
        e = pl.program_id(0)
        f = pl.program_id(1)
        my_y = lax.axis_index("y")
        ge = my_y * E_loc + e

        @pl.when(f == 0)
        def _():
            mask = a_ref[...].reshape(R, 1) == ge
            xm_ref[...] = jnp.where(mask, x_ref[...].reshape(R, D), jnp.bfloat16(0))

        w1 = w1_ref[0].astype(jnp.bfloat16)
        w2 = w2_ref[0].astype(jnp.bfloat16)
        h = jnp.dot(xm_ref[...], w1, preferred_element_type=jnp.float32)
        h = jnp.maximum(h, 0.0).astype(jnp.bfloat16)
        part = jnp.dot(h, w2, preferred_element_type=jnp.float32)

        first = jnp.logical_and(e == 0, f == 0)
        last = jnp.logical_and(e == E_loc - 1, f == n_f - 1)

        @pl.when(first)
        def _():
            acc_ref[...] = part

        @pl.when(jnp.logical_not(first))
        def _():
            acc_ref[...] += part

        @pl.when(last)
        def _():
            out_ref[...] = acc_ref[...].astype(jnp.bfloat16).reshape(n_m, S, D)

    return pl.pallas_call(
        body,
        grid=(E_loc, n_f),
        out_shape=jax.ShapeDtypeStruct((n_m, S, D), jnp.bfloat16),
        in_specs=[
            pl.BlockSpec((n_m, S, D), lambda e, f: (0, 0, 0)),
            pl.BlockSpec((n_m, S, 1), lambda e, f: (0, 0, 0)),
            pl.BlockSpec((1, D, FT), lambda e, f: (e, 0, f)),
            pl.BlockSpec((1, FT, D), lambda e, f: (e, f, 0)),
        ],
        out_specs=pl.BlockSpec((n_m, S, D), lambda e, f: (0, 0, 0)),
        scratch_shapes=[
            pltpu.VMEM((R, D), jnp.float32),
            pltpu.VMEM((R, D), jnp.bfloat16),
        ],
        compiler_params=pltpu.CompilerParams(
            dimension_semantics=("arbitrary", "arbitrary"),
            vmem_limit_bytes=62 * 2**20,
        ),
    )(xall, aall, W1f, W2f)


def _combine_allgather_kernel(contrib):
    _, S, D = contrib.shape
    n_r = N_RING // 2
    n_l = N_RING - 1 - n_r

    def body(
        c_ref,
        out_ref,
        own_ref,
        crecv_ref,
        commr_ref,
        comml_ref,
        copy_sem,
        c_send,
        c_recv,
        sr_send,
        sr_recv,
        sl_send,
        sl_recv,
    ):
        rp = _ring_pos()
        my_y = lax.axis_index("y")
        peer = _peer()
        rx, rz = _ring_coords((rp + 1) % N_RING)
        lx, lz = _ring_coords((rp - 1) % N_RING)
        right = (rx, my_y, rz)
        left = (lx, my_y, lz)

        local = pltpu.make_async_copy(c_ref.at[0], own_ref, copy_sem)
        local.start()
        rdma_c = pltpu.make_async_remote_copy(
            src_ref=c_ref.at[1],
            dst_ref=crecv_ref,
            send_sem=c_send,
            recv_sem=c_recv,
            device_id=peer,
            device_id_type=pl.DeviceIdType.MESH,
        )
        rdma_c.start()
        local.wait()
        rdma_c.wait()
        own = own_ref[...] + crecv_ref[...]

        out_ref[pl.ds(rp * S, S), :] = own
        commr_ref[0] = own
        comml_ref[0] = own

        for h in range(n_r):
            sslot = h % 2
            rslot = (h + 1) % 2
            rdma_r = pltpu.make_async_remote_copy(
                src_ref=commr_ref.at[sslot],
                dst_ref=commr_ref.at[rslot],
                send_sem=sr_send.at[sslot],
                recv_sem=sr_recv.at[rslot],
                device_id=right,
                device_id_type=pl.DeviceIdType.MESH,
            )
            rdma_r.start()
            if h < n_l:
                rdma_l = pltpu.make_async_remote_copy(
                    src_ref=comml_ref.at[sslot],
                    dst_ref=comml_ref.at[rslot],
                    send_sem=sl_send.at[sslot],
                    recv_sem=sl_recv.at[rslot],
                    device_id=left,
                    device_id_type=pl.DeviceIdType.MESH,
                )
                rdma_l.start()
            rdma_r.wait()
            origin_r = (rp - h - 1) % N_RING
            out_ref[pl.ds(origin_r * S, S), :] = commr_ref[rslot]
            if h < n_l:
                rdma_l.wait()
                origin_l = (rp + h + 1) % N_RING
                out_ref[pl.ds(origin_l * S, S), :] = comml_ref[rslot]

    return pl.pallas_call(
        body,
        out_shape=jax.ShapeDtypeStruct((N_RING * S, D), jnp.bfloat16),
        in_specs=[pl.BlockSpec(memory_space=pltpu.MemorySpace.HBM)],
        out_specs=pl.BlockSpec(memory_space=pltpu.MemorySpace.VMEM),
        scratch_shapes=[
            pltpu.VMEM((S, D), jnp.bfloat16),
            pltpu.VMEM((S, D), jnp.bfloat16),
            pltpu.VMEM((2, S, D), jnp.bfloat16),
            pltpu.VMEM((2, S, D), jnp.bfloat16),
            pltpu.SemaphoreType.DMA,
            pltpu.SemaphoreType.DMA,
            pltpu.SemaphoreType.DMA,
            pltpu.SemaphoreType.DMA((2,)),
            pltpu.SemaphoreType.DMA((2,)),
            pltpu.SemaphoreType.DMA((2,)),
            pltpu.SemaphoreType.DMA((2,)),
        ],
        compiler_params=pltpu.CompilerParams(vmem_limit_bytes=48 * 2**20),
    )(contrib)


def kernel(x, assign, W1, W2):
    T, D = x.shape
    S = T // N_RING
    xb = x.astype(jnp.bfloat16)
    a2 = assign.reshape(T, 1)

    rp = _ring_pos()
    xsl = lax.dynamic_slice(xb, (rp * S, 0), (S, D))
    asl = lax.dynamic_slice(a2, (rp * S, 0), (S, 1))

    xloc, aloc = _exchange_kernel(xsl, asl)
    contrib = _moe_kernel(xloc, aloc, W1, W2)
    return _combine_allgather_kernel(contrib)


# device time: 352354 ns/iter; 1.0689x vs baseline; 1.0360x over previous
import jax
import jax.numpy as jnp
from jax import lax
from jax.experimental import pallas as pl
from jax.experimental.pallas import tpu as pltpu

N_RING = 8


def _peer():
    return (lax.axis_index("x"), 1 - lax.axis_index("y"), lax.axis_index("z"))


def _ring_pos():
    x = lax.axis_index("x")
    z = lax.axis_index("z")
    return jnp.where(x == 0, z, 2 * N_RING // 2 - 1 - z)


def _ring_coords(t):
    x = jnp.where(t < N_RING // 2, 0, 1)
    z = jnp.where(t < N_RING // 2, t, N_RING - 1 - t)
    return x, z


def _exchange_kernel(xsl, asl):
    S, D = xsl.shape

    def body(x_ref, a_ref, xloc_ref, aloc_ref, sx_send, sx_recv, sa_send, sa_recv):
        peer = _peer()
        barrier = pltpu.get_barrier_semaphore()
        pl.semaphore_signal(
            barrier, inc=1, device_id=peer, device_id_type=pl.DeviceIdType.MESH
        )
        pl.semaphore_wait(barrier, 1)
        xloc_ref[0] = x_ref[...]
        aloc_ref[0] = a_ref[...]
        rx = pltpu.make_async_remote_copy(
            src_ref=x_ref,
            dst_ref=xloc_ref.at[1],
            send_sem=sx_send,
            recv_sem=sx_recv,
            device_id=peer,
            device_id_type=pl.DeviceIdType.MESH,
        )
        ra = pltpu.make_async_remote_copy(
            src_ref=a_ref,
            dst_ref=aloc_ref.at[1],
            send_sem=sa_send,
            recv_sem=sa_recv,
            device_id=peer,
            device_id_type=pl.DeviceIdType.MESH,
        )
        rx.start()
        ra.start()
        rx.wait()
        ra.wait()

    return pl.pallas_call(
        body,
        out_shape=(
            jax.ShapeDtypeStruct((2, S, D), jnp.bfloat16),
            jax.ShapeDtypeStruct((2, S, 1), jnp.int32),
        ),
        in_specs=[
            pl.BlockSpec(memory_space=pltpu.MemorySpace.VMEM),
            pl.BlockSpec(memory_space=pltpu.MemorySpace.VMEM),
        ],
        out_specs=(
            pl.BlockSpec(memory_space=pltpu.MemorySpace.VMEM),
            pl.BlockSpec(memory_space=pltpu.MemorySpace.VMEM),
        ),
        scratch_shapes=[
            pltpu.SemaphoreType.DMA,
            pltpu.SemaphoreType.DMA,
            pltpu.SemaphoreType.DMA,
            pltpu.SemaphoreType.DMA,
        ],
        compiler_params=pltpu.CompilerParams(collective_id=0),
    )(xsl, asl)


def _moe_kernel(xall, aall, W1f, W2f, FT=1024):
    n_m, S, D = xall.shape
    E_loc, _, F = W1f.shape
    n_f = F // FT
    R = n_m * S

    def body(x_ref, a_ref, w1_ref, w2_ref, out_ref, acc_ref, xm_ref, out_sem):
        e = pl.program_id(0)
        f = pl.program_id(1)
        my_y = lax.axis_index("y")
        ge = my_y * E_loc + e

        @pl.when(f == 0)
        def _():
            mask = a_ref[...].reshape(R, 1) == ge
            xm_ref[...] = jnp.where(mask, x_ref[...].reshape(R, D), jnp.bfloat16(0))

        w1 = w1_ref[0].astype(jnp.bfloat16)
        w2 = w2_ref[0].astype(jnp.bfloat16)
        h = jnp.dot(xm_ref[...], w1, preferred_element_type=jnp.float32)
        h = jnp.maximum(h, 0.0).astype(jnp.bfloat16)
        part = jnp.dot(h, w2, preferred_element_type=jnp.float32)

        first = jnp.logical_and(e == 0, f == 0)
        last = jnp.logical_and(e == E_loc - 1, f == n_f - 1)

        @pl.when(first)
        def _():
            acc_ref[...] = part

        @pl.when(jnp.logical_not(first))
        def _():
            acc_ref[...] += part

        @pl.when(last)
        def _():
            xm_ref[...] = acc_ref[...].astype(jnp.bfloat16)
            st = pltpu.make_async_copy(xm_ref, out_ref, out_sem)
            st.start()
            st.wait()

    return pl.pallas_call(
        body,
        grid=(E_loc, n_f),
        out_shape=jax.ShapeDtypeStruct((R, D), jnp.bfloat16),
        in_specs=[
            pl.BlockSpec((n_m, S, D), lambda e, f: (0, 0, 0)),
            pl.BlockSpec((n_m, S, 1), lambda e, f: (0, 0, 0)),
            pl.BlockSpec((1, D, FT), lambda e, f: (e, 0, f)),
            pl.BlockSpec((1, FT, D), lambda e, f: (e, f, 0)),
        ],
        out_specs=pl.BlockSpec(memory_space=pltpu.MemorySpace.HBM),
        scratch_shapes=[
            pltpu.VMEM((R, D), jnp.float32),
            pltpu.VMEM((R, D), jnp.bfloat16),
            pltpu.SemaphoreType.DMA,
        ],
        compiler_params=pltpu.CompilerParams(
            dimension_semantics=("arbitrary", "arbitrary"),
            vmem_limit_bytes=62 * 2**20,
        ),
    )(xall, aall, W1f, W2f)


def _combine_allgather_kernel(contrib):
    R, D = contrib.shape
    S = R // 2
    n_r = N_RING // 2
    n_l = N_RING - 1 - n_r

    def body(
        c_ref,
        out_ref,
        own_ref,
        crecv_ref,
        commr_ref,
        comml_ref,
        copy_sem,
        c_send,
        c_recv,
        sr_send,
        sr_recv,
        sl_send,
        sl_recv,
    ):
        rp = _ring_pos()
        my_y = lax.axis_index("y")
        peer = _peer()
        rx, rz = _ring_coords((rp + 1) % N_RING)
        lx, lz = _ring_coords((rp - 1) % N_RING)
        right = (rx, my_y, rz)
        left = (lx, my_y, lz)

        barrier = pltpu.get_barrier_semaphore()
        for nbr in (peer, right, left):
            pl.semaphore_signal(
                barrier, inc=1, device_id=nbr, device_id_type=pl.DeviceIdType.MESH
            )
        pl.semaphore_wait(barrier, 3)

        local = pltpu.make_async_copy(c_ref.at[pl.ds(0, S)], own_ref, copy_sem)
        local.start()
        rdma_c = pltpu.make_async_remote_copy(
            src_ref=c_ref.at[pl.ds(S, S)],
            dst_ref=crecv_ref,
            send_sem=c_send,
            recv_sem=c_recv,
            device_id=peer,
            device_id_type=pl.DeviceIdType.MESH,
        )
        rdma_c.start()
        local.wait()
        rdma_c.wait()
        own = own_ref[...] + crecv_ref[...]

        out_ref[pl.ds(rp * S, S), :] = own
        commr_ref[0] = own
        comml_ref[0] = own

        for h in range(n_r):
            sslot = h % 2
            rslot = (h + 1) % 2
            rdma_r = pltpu.make_async_remote_copy(
                src_ref=commr_ref.at[sslot],
                dst_ref=commr_ref.at[rslot],
                send_sem=sr_send.at[sslot],
                recv_sem=sr_recv.at[rslot],
                device_id=right,
                device_id_type=pl.DeviceIdType.MESH,
            )
            rdma_r.start()
            if h < n_l:
                rdma_l = pltpu.make_async_remote_copy(
                    src_ref=comml_ref.at[sslot],
                    dst_ref=comml_ref.at[rslot],
                    send_sem=sl_send.at[sslot],
                    recv_sem=sl_recv.at[rslot],
                    device_id=left,
                    device_id_type=pl.DeviceIdType.MESH,
                )
                rdma_l.start()
            rdma_r.wait()
            origin_r = (rp - h - 1) % N_RING
            out_ref[pl.ds(origin_r * S, S), :] = commr_ref[rslot]
            if h < n_l:
                rdma_l.wait()
                origin_l = (rp + h + 1) % N_RING
                out_ref[pl.ds(origin_l * S, S), :] = comml_ref[rslot]

    return pl.pallas_call(
        body,
        out_shape=jax.ShapeDtypeStruct((N_RING * S, D), jnp.bfloat16),
        in_specs=[pl.BlockSpec(memory_space=pltpu.MemorySpace.HBM)],
        out_specs=pl.BlockSpec(memory_space=pltpu.MemorySpace.VMEM),
        scratch_shapes=[
            pltpu.VMEM((S, D), jnp.bfloat16),
            pltpu.VMEM((S, D), jnp.bfloat16),
            pltpu.VMEM((2, S, D), jnp.bfloat16),
            pltpu.VMEM((2, S, D), jnp.bfloat16),
            pltpu.SemaphoreType.DMA,
            pltpu.SemaphoreType.DMA,
            pltpu.SemaphoreType.DMA,
            pltpu.SemaphoreType.DMA((2,)),
            pltpu.SemaphoreType.DMA((2,)),
            pltpu.SemaphoreType.DMA((2,)),
            pltpu.SemaphoreType.DMA((2,)),
        ],
        compiler_params=pltpu.CompilerParams(
            vmem_limit_bytes=48 * 2**20, collective_id=1
        ),
    )(contrib)


def kernel(x, assign, W1, W2):
    T, D = x.shape
    S = T // N_RING
    xb = x.astype(jnp.bfloat16)
    a2 = assign.reshape(T, 1)

    rp = _ring_pos()
    xsl = lax.dynamic_slice(xb, (rp * S, 0), (S, D))
    asl = lax.dynamic_slice(a2, (rp * S, 0), (S, 1))

    xloc, aloc = _exchange_kernel(xsl, asl)
    contrib = _moe_kernel(xloc, aloc, W1, W2)
    return _combine_allgather_kernel(contrib)
